# baseline (speedup 1.0000x reference)
import functools

import jax
import jax.numpy as jnp
from jax import lax
from jax.experimental import pallas as pl
from jax.experimental.pallas import tpu as pltpu
from jax.experimental.pallas import tpu_sc as plsc

Q_MIN = 0.1
EPS = 1e-09
N = 50000
D = 8
KP = 512
B = 2000
TB = N // B
BIG_I32 = 2**30

NWORK = 32
BINS = KP + 16
DUMMY = KP
CH = 1568
PAD_N = NWORK * CH
NCH = CH // 16

_SC_MESH = plsc.VectorSubcoreMesh(
    core_axis_name="c", subcore_axis_name="s", num_cores=2, num_subcores=16)


def _phase0_body(beta_ref, w_ref, q_ref, a_ref):
    b = beta_ref[...]
    q = 0.5 * jnp.log((1.0 + b) / (1.0 - b))
    q = q * q + Q_MIN
    q_ref[...] = q
    a_ref[...] = w_ref[...] * q


@functools.partial(
    pl.kernel,
    out_type=jax.ShapeDtypeStruct((NWORK * KP,), jnp.float32),
    mesh=_SC_MESH,
    compiler_params=pltpu.CompilerParams(needs_layout_passes=False),
    scratch_types=[
        pltpu.VMEM((CH,), jnp.float32),
        pltpu.VMEM((CH,), jnp.int32),
        pltpu.VMEM((BINS,), jnp.float32),
    ],
)
def _sc_stage_a(q_hbm, id_hbm, qmax_out, q_v, id_v, qm_v):
    wid = lax.axis_index("s") * 2 + lax.axis_index("c")
    base = wid * CH
    pltpu.sync_copy(q_hbm.at[pl.ds(base, CH)], q_v)
    pltpu.sync_copy(id_hbm.at[pl.ds(base, CH)], id_v)

    neg1 = jnp.full((16,), -1.0, jnp.float32)

    def init(i, carry):
        qm_v[pl.ds(i * 16, 16)] = neg1
        return carry
    lax.fori_loop(0, BINS // 16, init, 0)

    def chunk(c, carry):
        sl = pl.ds(c * 16, 16)
        ids_raw = id_v[sl]
        qs = q_v[sl]
        valid = ids_raw >= 0
        ids = jnp.where(valid, ids_raw, DUMMY)
        cur = plsc.load_gather(qm_v, [ids])
        win = valid & (qs > cur)

        def cond(w):
            return jnp.any(w)

        def body(w):
            idw = jnp.where(w, ids, DUMMY)
            plsc.store_scatter(qm_v, [idw], qs)
            cur2 = plsc.load_gather(qm_v, [ids])
            return w & (qs > cur2)

        lax.while_loop(cond, body, win)
        return carry
    lax.fori_loop(0, NCH, chunk, 0)

    pltpu.sync_copy(qm_v.at[pl.ds(0, KP)], qmax_out.at[pl.ds(wid * KP, KP)])


@functools.partial(
    pl.kernel,
    out_type=jax.ShapeDtypeStruct((NWORK * KP,), jnp.int32),
    mesh=_SC_MESH,
    compiler_params=pltpu.CompilerParams(needs_layout_passes=False),
    scratch_types=[
        pltpu.VMEM((CH,), jnp.float32),
        pltpu.VMEM((CH,), jnp.int32),
        pltpu.VMEM((NWORK * KP,), jnp.float32),
        pltpu.VMEM((BINS,), jnp.float32),
        pltpu.VMEM((BINS,), jnp.int32),
    ],
)
def _sc_stage_b(q_hbm, id_hbm, qmax_p_hbm, idx_out, q_v, id_v, qp_v, qg_v,
                ix_v):
    wid = lax.axis_index("s") * 2 + lax.axis_index("c")
    base = wid * CH
    pltpu.sync_copy(q_hbm.at[pl.ds(base, CH)], q_v)
    pltpu.sync_copy(id_hbm.at[pl.ds(base, CH)], id_v)
    pltpu.sync_copy(qmax_p_hbm, qp_v)

    neg1 = jnp.full((16,), -1.0, jnp.float32)
    big = jnp.full((16,), BIG_I32, jnp.int32)

    def init(i, carry):
        sl = pl.ds(i * 16, 16)
        qg_v[sl] = neg1
        ix_v[sl] = big
        return carry
    lax.fori_loop(0, BINS // 16, init, 0)

    def comb_row(r, carry):
        def comb(c, carry2):
            sl = pl.ds(c * 16, 16)
            qg_v[sl] = jnp.maximum(qg_v[sl], qp_v[pl.ds(r * KP + c * 16, 16)])
            return carry2
        lax.fori_loop(0, KP // 16, comb, 0)
        return carry
    lax.fori_loop(0, NWORK, comb_row, 0)

    lane = lax.iota(jnp.int32, 16)

    def chunk(c, carry):
        sl = pl.ds(c * 16, 16)
        ids_raw = id_v[sl]
        qs = q_v[sl]
        valid = ids_raw >= 0
        ids = jnp.where(valid, ids_raw, DUMMY)
        qmg = plsc.load_gather(qg_v, [ids])
        eq = valid & (qs == qmg)
        gi = lane + (base + c * 16)
        cur = plsc.load_gather(ix_v, [ids])
        win = eq & (gi < cur)

        def cond(w):
            return jnp.any(w)

        def body(w):
            idw = jnp.where(w, ids, DUMMY)
            plsc.store_scatter(ix_v, [idw], gi)
            cur2 = plsc.load_gather(ix_v, [ids])
            return w & (gi < cur2)

        lax.while_loop(cond, body, win)
        return carry
    lax.fori_loop(0, NCH, chunk, 0)

    pltpu.sync_copy(ix_v.at[pl.ds(0, KP)], idx_out.at[pl.ds(wid * KP, KP)])


@functools.partial(
    pl.kernel,
    out_type=jax.ShapeDtypeStruct((2 * KP,), jnp.float32),
    mesh=_SC_MESH,
    compiler_params=pltpu.CompilerParams(needs_layout_passes=False),
    scratch_types=[
        pltpu.VMEM((NWORK * KP,), jnp.float32),
        pltpu.VMEM((NWORK * KP,), jnp.int32),
        pltpu.VMEM((16,), jnp.float32),
    ],
)
def _sc_stage_c(qmax_p_hbm, idx_p_hbm, stats_out, qc_v, ic_v, tmp_v):
    wid = lax.axis_index("s") * 2 + lax.axis_index("c")
    pltpu.sync_copy(qmax_p_hbm, qc_v)
    pltpu.sync_copy(idx_p_hbm, ic_v)
    col = wid * 16

    def red(r, carry):
        qg, ig = carry
        sl = pl.ds(r * KP + col, 16)
        qg = jnp.maximum(qg, qc_v[sl])
        ig = jnp.minimum(ig, ic_v[sl])
        return qg, ig

    init = (
        jnp.full((16,), -1.0, jnp.float32),
        jnp.full((16,), BIG_I32, jnp.int32),
    )
    qg, ig = lax.fori_loop(0, NWORK, red, init)
    alpha = jnp.where(ig == BIG_I32, 0, ig).astype(jnp.float32)

    tmp_v[...] = alpha
    pltpu.sync_copy(tmp_v, stats_out.at[pl.ds(col, 16)])
    tmp_v[...] = qg
    pltpu.sync_copy(tmp_v, stats_out.at[pl.ds(KP + col, 16)])


def _phase2_body(alpha_ref, xf_ref, oid_ref, a_ref, beta_ref, stats_ref,
                 out_ref, xk_ref):
    def gather(k, carry):
        a = alpha_ref[k]
        xk_ref[pl.ds(k, 1), :] = xf_ref[pl.ds(a, 1), :]
        return carry
    lax.fori_loop(0, KP, gather, 0)

    qmax = stats_ref[1:2, :]

    xk = xk_ref[...]
    xkT = xk.T
    xkxk = jnp.sum(xkT * xkT, axis=0, keepdims=True)
    cand = lax.broadcasted_iota(jnp.int32, (B, KP), 1)

    def tile(t, carry):
        s_rep, s_att, cnt, nb = carry
        xt = xf_ref[pl.ds(t * B, B), :]
        ids = oid_ref[t][:, None]
        a = a_ref[t][:, None]
        xx = jnp.sum(xt * xt, axis=1)[:, None]
        cross = lax.dot_general(
            xt, xkT, (((1,), (0,)), ((), ())),
            preferred_element_type=jnp.float32)
        d2 = jnp.maximum(xx + xkxk - 2.0 * cross, 0.0)
        dist = jnp.sqrt(d2 + 1e-12)
        attm = ids == cand
        rep_e = jnp.where(attm, 0.0, jnp.maximum(1.0 - dist, 0.0))
        att_e = jnp.where(attm, d2, 0.0)
        s_rep = s_rep + jnp.sum(a * rep_e, axis=0, keepdims=True)
        s_att = s_att + jnp.sum(a * att_e, axis=0, keepdims=True)
        cnt = cnt + jnp.sum(attm.astype(jnp.float32), axis=0, keepdims=True)
        noise = ids == 0
        nb = nb + jnp.sum(jnp.where(noise, beta_ref[t][:, None], 0.0))
        return s_rep, s_att, cnt, nb

    z = jnp.zeros((1, KP), jnp.float32)
    s_rep, s_att, cnt, nb = lax.fori_loop(
        0, TB, tile, (z, z, z, jnp.float32(0.0)))

    colid = lax.broadcasted_iota(jnp.int32, (1, KP), 1)
    present = (cnt > 0.0) & (colid >= 1)
    k_f = jnp.sum(present.astype(jnp.float32))
    qk = jnp.where(present, qmax, 0.0)
    c_att = qk / ((cnt + EPS) * k_f)
    c_rep = qk / ((jnp.float32(N) - cnt + EPS) * k_f)

    v_att = jnp.sum(s_att * c_att)
    v_rep = jnp.sum(s_rep * c_rep)
    beta_k = jnp.tanh(jnp.sqrt(jnp.maximum(qmax - Q_MIN, 0.0)))
    l_cow = jnp.sum(jnp.where(present, 1.0 - beta_k, 0.0)) / k_f
    nc = cnt[0, 0]
    l_noise = nb / jnp.maximum(nc, 1.0)

    li = lax.broadcasted_iota(jnp.int32, (8, 128), 1)
    out = jnp.where(li == 0, v_att,
          jnp.where(li == 1, v_rep,
          jnp.where(li == 2, l_cow,
          jnp.where(li == 3, l_noise, 0.0))))
    out_ref[...] = out


@jax.jit
def kernel(beta, x, object_id, weights):
    beta2 = beta.reshape(TB, B)
    oid2 = object_id.reshape(TB, B)
    w2 = weights.reshape(TB, B)

    q2, a2 = pl.pallas_call(
        _phase0_body,
        out_shape=(
            jax.ShapeDtypeStruct((TB, B), jnp.float32),
            jax.ShapeDtypeStruct((TB, B), jnp.float32),
        ),
    )(beta2, w2)

    pad = PAD_N - N
    qflat = jnp.concatenate([q2.reshape(N), jnp.zeros((pad,), jnp.float32)])
    idflat = jnp.concatenate(
        [object_id, jnp.full((pad,), -1, jnp.int32)])

    qmax_p = _sc_stage_a(qflat, idflat)
    idx_p = _sc_stage_b(qflat, idflat, qmax_p)
    stats = _sc_stage_c(qmax_p, idx_p).reshape(2, KP)

    alphas = stats[0].astype(jnp.int32)

    out = pl.pallas_call(
        _phase2_body,
        out_shape=jax.ShapeDtypeStruct((8, 128), jnp.float32),
        in_specs=[
            pl.BlockSpec(memory_space=pltpu.MemorySpace.SMEM),
            pl.BlockSpec(memory_space=pltpu.MemorySpace.VMEM),
            pl.BlockSpec(memory_space=pltpu.MemorySpace.VMEM),
            pl.BlockSpec(memory_space=pltpu.MemorySpace.VMEM),
            pl.BlockSpec(memory_space=pltpu.MemorySpace.VMEM),
            pl.BlockSpec(memory_space=pltpu.MemorySpace.VMEM),
        ],
        scratch_shapes=[pltpu.VMEM((KP, D), jnp.float32)],
    )(alphas, x, oid2, a2, beta2, stats)

    return (out[0, 0], out[0, 1], out[0, 2], out[0, 3])


# speedup vs baseline: 1.1412x; 1.1412x over previous
import functools

import jax
import jax.numpy as jnp
from jax import lax
from jax.experimental import pallas as pl
from jax.experimental.pallas import tpu as pltpu
from jax.experimental.pallas import tpu_sc as plsc

Q_MIN = 0.1
EPS = 1e-09
N = 50000
D = 8
KP = 512
B = 2000
TB = N // B
BIG_I32 = 2**30

NWORK = 32
BINS = KP + 16
DUMMY = KP
CH = 1568
PAD_N = NWORK * CH
NCH = CH // 16

_SC_MESH = plsc.VectorSubcoreMesh(
    core_axis_name="c", subcore_axis_name="s", num_cores=2, num_subcores=16)
_SC_PARAMS = pltpu.CompilerParams(needs_layout_passes=False)


def _phase0_body(beta_ref, w_ref, q_ref, a_ref):
    b = beta_ref[...]
    q = 0.5 * jnp.log((1.0 + b) / (1.0 - b))
    q = q * q + Q_MIN
    q_ref[...] = q
    a_ref[...] = w_ref[...] * q


@functools.partial(
    pl.kernel,
    out_type=(
        jax.ShapeDtypeStruct((NWORK * KP,), jnp.float32),
        jax.ShapeDtypeStruct((NWORK * KP,), jnp.float32),
        jax.ShapeDtypeStruct((NWORK * KP,), jnp.float32),
    ),
    mesh=_SC_MESH,
    compiler_params=_SC_PARAMS,
    scratch_types=[
        pltpu.VMEM((CH,), jnp.float32),
        pltpu.VMEM((CH,), jnp.int32),
        pltpu.VMEM((CH,), jnp.float32),
        pltpu.VMEM((BINS,), jnp.float32),
        pltpu.VMEM((BINS,), jnp.float32),
        pltpu.VMEM((BINS,), jnp.float32),
    ],
)
def _sc_stage_a(q_hbm, id_hbm, beta_hbm, qmax_out, cnt_out, bsum_out,
                q_v, id_v, b_v, qm_v, ct_v, bs_v):
    wid = lax.axis_index("s") * 2 + lax.axis_index("c")
    base = wid * CH
    pltpu.sync_copy(q_hbm.at[pl.ds(base, CH)], q_v)
    pltpu.sync_copy(id_hbm.at[pl.ds(base, CH)], id_v)
    pltpu.sync_copy(beta_hbm.at[pl.ds(base, CH)], b_v)

    neg1 = jnp.full((16,), -1.0, jnp.float32)
    zero = jnp.zeros((16,), jnp.float32)

    def init(i, carry):
        sl = pl.ds(i * 16, 16)
        qm_v[sl] = neg1
        ct_v[sl] = zero
        bs_v[sl] = zero
        return carry
    lax.fori_loop(0, BINS // 16, init, 0)

    ones = jnp.ones((16,), jnp.float32)

    def chunk(c, carry):
        sl = pl.ds(c * 16, 16)
        ids_raw = id_v[sl]
        qs = q_v[sl]
        bs = b_v[sl]
        valid = ids_raw >= 0
        ids = jnp.where(valid, ids_raw, DUMMY)
        plsc.addupdate_scatter(ct_v, [ids], jnp.where(valid, ones, 0.0))
        plsc.addupdate_scatter(bs_v, [ids], jnp.where(valid, bs, 0.0))
        cur = plsc.load_gather(qm_v, [ids])
        win = valid & (qs > cur)

        def cond(w):
            return jnp.any(w)

        def body(w):
            idw = jnp.where(w, ids, DUMMY)
            plsc.store_scatter(qm_v, [idw], qs)
            cur2 = plsc.load_gather(qm_v, [ids])
            return w & (qs > cur2)

        lax.while_loop(cond, body, win)
        return carry
    lax.fori_loop(0, NCH, chunk, 0)

    pltpu.sync_copy(qm_v.at[pl.ds(0, KP)], qmax_out.at[pl.ds(wid * KP, KP)])
    pltpu.sync_copy(ct_v.at[pl.ds(0, KP)], cnt_out.at[pl.ds(wid * KP, KP)])
    pltpu.sync_copy(bs_v.at[pl.ds(0, KP)], bsum_out.at[pl.ds(wid * KP, KP)])


def _comb1_body(qp_ref, cp_ref, bp_ref, stats_ref):
    stats_ref[0:1, :] = jnp.zeros((1, KP), jnp.float32)
    stats_ref[1:2, :] = jnp.sum(cp_ref[...], axis=0, keepdims=True)
    stats_ref[2:3, :] = jnp.max(qp_ref[...], axis=0, keepdims=True)
    stats_ref[3:4, :] = jnp.sum(bp_ref[...], axis=0, keepdims=True)
    stats_ref[4:8, :] = jnp.zeros((4, KP), jnp.float32)


@functools.partial(
    pl.kernel,
    out_type=jax.ShapeDtypeStruct((NWORK * KP,), jnp.int32),
    mesh=_SC_MESH,
    compiler_params=_SC_PARAMS,
    scratch_types=[
        pltpu.VMEM((CH,), jnp.float32),
        pltpu.VMEM((CH,), jnp.int32),
        pltpu.VMEM((BINS,), jnp.float32),
        pltpu.VMEM((BINS,), jnp.int32),
    ],
)
def _sc_stage_b(q_hbm, id_hbm, qmaxg_hbm, idx_out, q_v, id_v, qg_v, ix_v):
    wid = lax.axis_index("s") * 2 + lax.axis_index("c")
    base = wid * CH
    pltpu.sync_copy(q_hbm.at[pl.ds(base, CH)], q_v)
    pltpu.sync_copy(id_hbm.at[pl.ds(base, CH)], id_v)
    pltpu.sync_copy(qmaxg_hbm, qg_v.at[pl.ds(0, KP)])

    neg1 = jnp.full((16,), -1.0, jnp.float32)
    big = jnp.full((16,), BIG_I32, jnp.int32)
    qg_v[pl.ds(KP, 16)] = neg1

    def init(i, carry):
        ix_v[pl.ds(i * 16, 16)] = big
        return carry
    lax.fori_loop(0, BINS // 16, init, 0)

    lane = lax.iota(jnp.int32, 16)

    def chunk(c, carry):
        sl = pl.ds(c * 16, 16)
        ids_raw = id_v[sl]
        qs = q_v[sl]
        valid = ids_raw >= 0
        ids = jnp.where(valid, ids_raw, DUMMY)
        qmg = plsc.load_gather(qg_v, [ids])
        eq = valid & (qs == qmg)
        gi = lane + (base + c * 16)
        cur = plsc.load_gather(ix_v, [ids])
        win = eq & (gi < cur)

        def cond(w):
            return jnp.any(w)

        def body(w):
            idw = jnp.where(w, ids, DUMMY)
            plsc.store_scatter(ix_v, [idw], gi)
            cur2 = plsc.load_gather(ix_v, [ids])
            return w & (gi < cur2)

        lax.while_loop(cond, body, win)
        return carry
    lax.fori_loop(0, NCH, chunk, 0)

    pltpu.sync_copy(ix_v.at[pl.ds(0, KP)], idx_out.at[pl.ds(wid * KP, KP)])


def _comb2_body(ip_ref, alpha_ref):
    ig = jnp.min(ip_ref[...], axis=0, keepdims=True)
    a = jnp.where(ig == BIG_I32, 0, ig)
    alpha_ref[...] = jnp.broadcast_to(a, (8, KP))


def _phase2_body(alpha_ref, xf_ref, oid_ref, a_ref, stats_ref,
                 out_ref, xk_ref):
    def gather(k, carry):
        a = alpha_ref[k]
        xk_ref[pl.ds(k, 1), :] = xf_ref[pl.ds(a, 1), :]
        return carry
    lax.fori_loop(0, KP, gather, 0)

    cnt = stats_ref[1:2, :]
    qmax = stats_ref[2:3, :]
    nb = stats_ref[3, 0]
    nc = stats_ref[1, 0]

    colid = lax.broadcasted_iota(jnp.int32, (1, KP), 1)
    present = (cnt > 0.0) & (colid >= 1)
    k_f = jnp.sum(present.astype(jnp.float32))
    qk = jnp.where(present, qmax, 0.0)
    c_att = qk / ((cnt + EPS) * k_f)
    c_rep = qk / ((jnp.float32(N) - cnt + EPS) * k_f)

    xk = xk_ref[...]
    xkT = xk.T
    xkxk = jnp.sum(xkT * xkT, axis=0, keepdims=True)
    cand = lax.broadcasted_iota(jnp.int32, (B, KP), 1)

    def tile(t, carry):
        s_rep, s_att = carry
        xt = xf_ref[pl.ds(t * B, B), :]
        ids = oid_ref[t][:, None]
        a = a_ref[t][:, None]
        xx = jnp.sum(xt * xt, axis=1)[:, None]
        cross = lax.dot_general(
            xt, xkT, (((1,), (0,)), ((), ())),
            preferred_element_type=jnp.float32)
        d2 = jnp.maximum(xx + xkxk - 2.0 * cross, 0.0)
        dist = jnp.sqrt(d2 + 1e-12)
        attm = ids == cand
        rep_e = jnp.where(attm, 0.0, jnp.maximum(1.0 - dist, 0.0))
        att_e = jnp.where(attm, d2, 0.0)
        s_rep = s_rep + jnp.sum(a * rep_e, axis=0, keepdims=True)
        s_att = s_att + jnp.sum(a * att_e, axis=0, keepdims=True)
        return s_rep, s_att

    z = jnp.zeros((1, KP), jnp.float32)
    s_rep, s_att = lax.fori_loop(0, TB, tile, (z, z))

    v_att = jnp.sum(s_att * c_att)
    v_rep = jnp.sum(s_rep * c_rep)
    beta_k = jnp.tanh(jnp.sqrt(jnp.maximum(qmax - Q_MIN, 0.0)))
    l_cow = jnp.sum(jnp.where(present, 1.0 - beta_k, 0.0)) / k_f
    l_noise = nb / jnp.maximum(nc, 1.0)

    li = lax.broadcasted_iota(jnp.int32, (8, 128), 1)
    out = jnp.where(li == 0, v_att,
          jnp.where(li == 1, v_rep,
          jnp.where(li == 2, l_cow,
          jnp.where(li == 3, l_noise, 0.0))))
    out_ref[...] = out


@jax.jit
def kernel(beta, x, object_id, weights):
    beta2 = beta.reshape(TB, B)
    oid2 = object_id.reshape(TB, B)
    w2 = weights.reshape(TB, B)

    q2, a2 = pl.pallas_call(
        _phase0_body,
        out_shape=(
            jax.ShapeDtypeStruct((TB, B), jnp.float32),
            jax.ShapeDtypeStruct((TB, B), jnp.float32),
        ),
    )(beta2, w2)

    pad = PAD_N - N
    qflat = jnp.concatenate([q2.reshape(N), jnp.zeros((pad,), jnp.float32)])
    idflat = jnp.concatenate(
        [object_id, jnp.full((pad,), -1, jnp.int32)])
    bflat = jnp.concatenate([beta, jnp.zeros((pad,), jnp.float32)])

    qmax_p, cnt_p, bsum_p = _sc_stage_a(qflat, idflat, bflat)

    stats = pl.pallas_call(
        _comb1_body,
        out_shape=jax.ShapeDtypeStruct((8, KP), jnp.float32),
    )(qmax_p.reshape(NWORK, KP), cnt_p.reshape(NWORK, KP),
      bsum_p.reshape(NWORK, KP))

    qmaxg = stats[2]

    idx_p = _sc_stage_b(qflat, idflat, qmaxg)

    alpha8 = pl.pallas_call(
        _comb2_body,
        out_shape=jax.ShapeDtypeStruct((8, KP), jnp.int32),
    )(idx_p.reshape(NWORK, KP))
    alphas = alpha8[0]

    out = pl.pallas_call(
        _phase2_body,
        out_shape=jax.ShapeDtypeStruct((8, 128), jnp.float32),
        in_specs=[
            pl.BlockSpec(memory_space=pltpu.MemorySpace.SMEM),
            pl.BlockSpec(memory_space=pltpu.MemorySpace.VMEM),
            pl.BlockSpec(memory_space=pltpu.MemorySpace.VMEM),
            pl.BlockSpec(memory_space=pltpu.MemorySpace.VMEM),
            pl.BlockSpec(memory_space=pltpu.MemorySpace.VMEM),
        ],
        scratch_shapes=[pltpu.VMEM((KP, D), jnp.float32)],
    )(alphas, x, oid2, a2, stats)

    return (out[0, 0], out[0, 1], out[0, 2], out[0, 3])


# speedup vs baseline: 1.2761x; 1.1182x over previous
import functools

import jax
import jax.numpy as jnp
from jax import lax
from jax.experimental import pallas as pl
from jax.experimental.pallas import tpu as pltpu
from jax.experimental.pallas import tpu_sc as plsc

Q_MIN = 0.1
EPS = 1e-09
N = 50000
D = 8
KP = 512
B = 2000
TB = N // B
BIG_I32 = 2**30

NWORK = 32
BINS = KP + 16
DUMMY = KP
CH = 1568
PAD_N = NWORK * CH
NCH = CH // 16

_SC_MESH = plsc.VectorSubcoreMesh(
    core_axis_name="c", subcore_axis_name="s", num_cores=2, num_subcores=16)
_SC_PARAMS = pltpu.CompilerParams(needs_layout_passes=False)


def _phase0_body(beta_ref, w_ref, q_ref, a_ref):
    b = beta_ref[...]
    q = 0.5 * jnp.log((1.0 + b) / (1.0 - b))
    q = q * q + Q_MIN
    q_ref[...] = q
    a_ref[...] = w_ref[...] * q


@functools.partial(
    pl.kernel,
    out_type=(
        jax.ShapeDtypeStruct((NWORK * KP,), jnp.float32),
        jax.ShapeDtypeStruct((NWORK * KP,), jnp.float32),
        jax.ShapeDtypeStruct((NWORK * KP,), jnp.float32),
    ),
    mesh=_SC_MESH,
    compiler_params=_SC_PARAMS,
    scratch_types=[
        pltpu.VMEM((CH,), jnp.float32),
        pltpu.VMEM((CH,), jnp.int32),
        pltpu.VMEM((CH,), jnp.float32),
        pltpu.VMEM((BINS,), jnp.float32),
        pltpu.VMEM((BINS,), jnp.float32),
        pltpu.VMEM((BINS,), jnp.float32),
    ],
)
def _sc_stage_a(q_hbm, id_hbm, beta_hbm, qmax_out, cnt_out, bsum_out,
                q_v, id_v, b_v, qm_v, ct_v, bs_v):
    wid = lax.axis_index("s") * 2 + lax.axis_index("c")
    base = wid * CH
    pltpu.sync_copy(q_hbm.at[pl.ds(base, CH)], q_v)
    pltpu.sync_copy(id_hbm.at[pl.ds(base, CH)], id_v)
    pltpu.sync_copy(beta_hbm.at[pl.ds(base, CH)], b_v)

    neg1 = jnp.full((16,), -1.0, jnp.float32)
    zero = jnp.zeros((16,), jnp.float32)

    def init(i, carry):
        sl = pl.ds(i * 16, 16)
        qm_v[sl] = neg1
        ct_v[sl] = zero
        bs_v[sl] = zero
        return carry
    lax.fori_loop(0, BINS // 16, init, 0)

    ones = jnp.ones((16,), jnp.float32)

    def chunk(c, carry):
        sl = pl.ds(c * 16, 16)
        ids_raw = id_v[sl]
        qs = q_v[sl]
        bs = b_v[sl]
        valid = ids_raw >= 0
        ids = jnp.where(valid, ids_raw, DUMMY)
        plsc.addupdate_scatter(ct_v, [ids], jnp.where(valid, ones, 0.0))
        plsc.addupdate_scatter(bs_v, [ids], jnp.where(valid, bs, 0.0))
        cur = plsc.load_gather(qm_v, [ids])
        win = valid & (qs > cur)

        def cond(w):
            return jnp.any(w)

        def body(w):
            idw = jnp.where(w, ids, DUMMY)
            plsc.store_scatter(qm_v, [idw], qs)
            cur2 = plsc.load_gather(qm_v, [ids])
            return w & (qs > cur2)

        lax.while_loop(cond, body, win)
        return carry
    lax.fori_loop(0, NCH, chunk, 0)

    pltpu.sync_copy(qm_v.at[pl.ds(0, KP)], qmax_out.at[pl.ds(wid * KP, KP)])
    pltpu.sync_copy(ct_v.at[pl.ds(0, KP)], cnt_out.at[pl.ds(wid * KP, KP)])
    pltpu.sync_copy(bs_v.at[pl.ds(0, KP)], bsum_out.at[pl.ds(wid * KP, KP)])


def _comb1_body(qp_ref, cp_ref, bp_ref, stats_ref):
    stats_ref[0:1, :] = jnp.zeros((1, KP), jnp.float32)
    stats_ref[1:2, :] = jnp.sum(cp_ref[...], axis=0, keepdims=True)
    stats_ref[2:3, :] = jnp.max(qp_ref[...], axis=0, keepdims=True)
    stats_ref[3:4, :] = jnp.sum(bp_ref[...], axis=0, keepdims=True)
    stats_ref[4:8, :] = jnp.zeros((4, KP), jnp.float32)


@functools.partial(
    pl.kernel,
    out_type=jax.ShapeDtypeStruct((NWORK * KP,), jnp.int32),
    mesh=_SC_MESH,
    compiler_params=_SC_PARAMS,
    scratch_types=[
        pltpu.VMEM((CH,), jnp.float32),
        pltpu.VMEM((CH,), jnp.int32),
        pltpu.VMEM((BINS,), jnp.float32),
        pltpu.VMEM((BINS,), jnp.int32),
    ],
)
def _sc_stage_b(q_hbm, id_hbm, qmaxg_hbm, idx_out, q_v, id_v, qg_v, ix_v):
    wid = lax.axis_index("s") * 2 + lax.axis_index("c")
    base = wid * CH
    pltpu.sync_copy(q_hbm.at[pl.ds(base, CH)], q_v)
    pltpu.sync_copy(id_hbm.at[pl.ds(base, CH)], id_v)
    pltpu.sync_copy(qmaxg_hbm, qg_v.at[pl.ds(0, KP)])

    neg1 = jnp.full((16,), -1.0, jnp.float32)
    big = jnp.full((16,), BIG_I32, jnp.int32)
    qg_v[pl.ds(KP, 16)] = neg1

    def init(i, carry):
        ix_v[pl.ds(i * 16, 16)] = big
        return carry
    lax.fori_loop(0, BINS // 16, init, 0)

    lane = lax.iota(jnp.int32, 16)

    def chunk(c, carry):
        sl = pl.ds(c * 16, 16)
        ids_raw = id_v[sl]
        qs = q_v[sl]
        valid = ids_raw >= 0
        ids = jnp.where(valid, ids_raw, DUMMY)
        qmg = plsc.load_gather(qg_v, [ids])
        eq = valid & (qs == qmg)
        gi = lane + (base + c * 16)
        cur = plsc.load_gather(ix_v, [ids])
        win = eq & (gi < cur)

        def cond(w):
            return jnp.any(w)

        def body(w):
            idw = jnp.where(w, ids, DUMMY)
            plsc.store_scatter(ix_v, [idw], gi)
            cur2 = plsc.load_gather(ix_v, [ids])
            return w & (gi < cur2)

        lax.while_loop(cond, body, win)
        return carry
    lax.fori_loop(0, NCH, chunk, 0)

    pltpu.sync_copy(ix_v.at[pl.ds(0, KP)], idx_out.at[pl.ds(wid * KP, KP)])


def _comb2_body(ip_ref, alpha_ref):
    ig = jnp.min(ip_ref[...], axis=0, keepdims=True)
    a = jnp.where(ig == BIG_I32, 0, ig)
    alpha_ref[...] = jnp.broadcast_to(a, (8, KP))


def _phase2_body(alpha_ref, xf_ref, oid_ref, a_ref, stats_ref,
                 out_ref, xk_ref):
    def gather(k, carry):
        a = alpha_ref[k]
        xk_ref[pl.ds(k, 1), :] = xf_ref[pl.ds(a, 1), :]
        return carry
    lax.fori_loop(0, KP, gather, 0)

    cnt = stats_ref[1:2, :]
    qmax = stats_ref[2:3, :]
    nb = stats_ref[3, 0]
    nc = stats_ref[1, 0]

    colid = lax.broadcasted_iota(jnp.int32, (1, KP), 1)
    present = (cnt > 0.0) & (colid >= 1)
    k_f = jnp.sum(present.astype(jnp.float32))
    qk = jnp.where(present, qmax, 0.0)
    c_att = qk / ((cnt + EPS) * k_f)
    c_rep = qk / ((jnp.float32(N) - cnt + EPS) * k_f)

    xk = xk_ref[...]
    xkT = xk.T
    xkxk = jnp.sum(xkT * xkT, axis=0, keepdims=True)
    xkT2 = -2.0 * xkT
    cand = lax.broadcasted_iota(jnp.int32, (B, KP), 1)

    def tile(t, carry):
        s_rep, s_att = carry
        xt = xf_ref[pl.ds(t * B, B), :]
        ids = oid_ref[t][:, None]
        a = a_ref[t][:, None]
        xx = jnp.sum(xt * xt, axis=1)[:, None]
        cross = lax.dot_general(
            xt, xkT2, (((1,), (0,)), ((), ())),
            preferred_element_type=jnp.float32)
        d2 = jnp.maximum(xx + (xkxk + cross), 0.0)
        d2p = d2 + 1e-12
        dist = d2p * lax.rsqrt(d2p)
        attm = ids == cand
        rep_e = jnp.where(attm, 0.0, jnp.maximum(1.0 - dist, 0.0))
        att_e = jnp.where(attm, d2, 0.0)
        s_rep = s_rep + jnp.sum(a * rep_e, axis=0, keepdims=True)
        s_att = s_att + jnp.sum(a * att_e, axis=0, keepdims=True)
        return s_rep, s_att

    z = jnp.zeros((1, KP), jnp.float32)
    s_rep, s_att = lax.fori_loop(0, TB, tile, (z, z))

    v_att = jnp.sum(s_att * c_att)
    v_rep = jnp.sum(s_rep * c_rep)
    beta_k = jnp.tanh(jnp.sqrt(jnp.maximum(qmax - Q_MIN, 0.0)))
    l_cow = jnp.sum(jnp.where(present, 1.0 - beta_k, 0.0)) / k_f
    l_noise = nb / jnp.maximum(nc, 1.0)

    li = lax.broadcasted_iota(jnp.int32, (8, 128), 1)
    out = jnp.where(li == 0, v_att,
          jnp.where(li == 1, v_rep,
          jnp.where(li == 2, l_cow,
          jnp.where(li == 3, l_noise, 0.0))))
    out_ref[...] = out


@jax.jit
def kernel(beta, x, object_id, weights):
    beta2 = beta.reshape(TB, B)
    oid2 = object_id.reshape(TB, B)
    w2 = weights.reshape(TB, B)

    q2, a2 = pl.pallas_call(
        _phase0_body,
        out_shape=(
            jax.ShapeDtypeStruct((TB, B), jnp.float32),
            jax.ShapeDtypeStruct((TB, B), jnp.float32),
        ),
    )(beta2, w2)

    pad = PAD_N - N
    qflat = jnp.concatenate([q2.reshape(N), jnp.zeros((pad,), jnp.float32)])
    idflat = jnp.concatenate(
        [object_id, jnp.full((pad,), -1, jnp.int32)])
    bflat = jnp.concatenate([beta, jnp.zeros((pad,), jnp.float32)])

    qmax_p, cnt_p, bsum_p = _sc_stage_a(qflat, idflat, bflat)

    stats = pl.pallas_call(
        _comb1_body,
        out_shape=jax.ShapeDtypeStruct((8, KP), jnp.float32),
    )(qmax_p.reshape(NWORK, KP), cnt_p.reshape(NWORK, KP),
      bsum_p.reshape(NWORK, KP))

    qmaxg = stats[2]

    idx_p = _sc_stage_b(qflat, idflat, qmaxg)

    alpha8 = pl.pallas_call(
        _comb2_body,
        out_shape=jax.ShapeDtypeStruct((8, KP), jnp.int32),
    )(idx_p.reshape(NWORK, KP))
    alphas = alpha8[0]

    out = pl.pallas_call(
        _phase2_body,
        out_shape=jax.ShapeDtypeStruct((8, 128), jnp.float32),
        in_specs=[
            pl.BlockSpec(memory_space=pltpu.MemorySpace.SMEM),
            pl.BlockSpec(memory_space=pltpu.MemorySpace.VMEM),
            pl.BlockSpec(memory_space=pltpu.MemorySpace.VMEM),
            pl.BlockSpec(memory_space=pltpu.MemorySpace.VMEM),
            pl.BlockSpec(memory_space=pltpu.MemorySpace.VMEM),
        ],
        scratch_shapes=[pltpu.VMEM((KP, D), jnp.float32)],
    )(alphas, x, oid2, a2, stats)

    return (out[0, 0], out[0, 1], out[0, 2], out[0, 3])


# speedup vs baseline: 1.3053x; 1.0229x over previous
import functools

import jax
import jax.numpy as jnp
from jax import lax
from jax.experimental import pallas as pl
from jax.experimental.pallas import tpu as pltpu
from jax.experimental.pallas import tpu_sc as plsc

Q_MIN = 0.1
EPS = 1e-09
N = 50000
D = 8
KP = 512
B = 2000
TB = N // B
BIG_I32 = 2**30

NWORK = 32
BINS = KP + 16
DUMMY = KP
CH = 1568
PAD_N = NWORK * CH
NCH = CH // 16

_SC_MESH = plsc.VectorSubcoreMesh(
    core_axis_name="c", subcore_axis_name="s", num_cores=2, num_subcores=16)
_SC_PARAMS = pltpu.CompilerParams(needs_layout_passes=False)


def _phase0_body(beta_ref, w_ref, q_ref, a_ref):
    b = beta_ref[...]
    q = 0.5 * jnp.log((1.0 + b) / (1.0 - b))
    q = q * q + Q_MIN
    q_ref[...] = q
    a_ref[...] = w_ref[...] * q


@functools.partial(
    pl.kernel,
    out_type=(
        jax.ShapeDtypeStruct((NWORK * KP,), jnp.float32),
        jax.ShapeDtypeStruct((NWORK * KP,), jnp.float32),
        jax.ShapeDtypeStruct((NWORK * KP,), jnp.float32),
    ),
    mesh=_SC_MESH,
    compiler_params=_SC_PARAMS,
    scratch_types=[
        pltpu.VMEM((CH,), jnp.float32),
        pltpu.VMEM((CH,), jnp.int32),
        pltpu.VMEM((CH,), jnp.float32),
        pltpu.VMEM((BINS,), jnp.float32),
        pltpu.VMEM((BINS,), jnp.float32),
        pltpu.VMEM((BINS,), jnp.float32),
    ],
)
def _sc_stage_a(q_hbm, id_hbm, beta_hbm, qmax_out, cnt_out, bsum_out,
                q_v, id_v, b_v, qm_v, ct_v, bs_v):
    wid = lax.axis_index("s") * 2 + lax.axis_index("c")
    base = wid * CH
    pltpu.sync_copy(q_hbm.at[pl.ds(base, CH)], q_v)
    pltpu.sync_copy(id_hbm.at[pl.ds(base, CH)], id_v)
    pltpu.sync_copy(beta_hbm.at[pl.ds(base, CH)], b_v)

    neg1 = jnp.full((16,), -1.0, jnp.float32)
    zero = jnp.zeros((16,), jnp.float32)

    def init(i, carry):
        sl = pl.ds(i * 16, 16)
        qm_v[sl] = neg1
        ct_v[sl] = zero
        bs_v[sl] = zero
        return carry
    lax.fori_loop(0, BINS // 16, init, 0)

    ones = jnp.ones((16,), jnp.float32)

    def chunk(c, carry):
        sl = pl.ds(c * 16, 16)
        ids_raw = id_v[sl]
        qs = q_v[sl]
        bs = b_v[sl]
        valid = ids_raw >= 0
        ids = jnp.where(valid, ids_raw, DUMMY)
        plsc.addupdate_scatter(ct_v, [ids], jnp.where(valid, ones, 0.0))
        plsc.addupdate_scatter(bs_v, [ids], jnp.where(valid, bs, 0.0))
        cur = plsc.load_gather(qm_v, [ids])
        win = valid & (qs > cur)

        def cond(w):
            return jnp.any(w)

        def body(w):
            idw = jnp.where(w, ids, DUMMY)
            plsc.store_scatter(qm_v, [idw], qs)
            cur2 = plsc.load_gather(qm_v, [ids])
            return w & (qs > cur2)

        lax.while_loop(cond, body, win)
        return carry
    lax.fori_loop(0, NCH, chunk, 0)

    pltpu.sync_copy(qm_v.at[pl.ds(0, KP)], qmax_out.at[pl.ds(wid * KP, KP)])
    pltpu.sync_copy(ct_v.at[pl.ds(0, KP)], cnt_out.at[pl.ds(wid * KP, KP)])
    pltpu.sync_copy(bs_v.at[pl.ds(0, KP)], bsum_out.at[pl.ds(wid * KP, KP)])


def _comb1_body(qp_ref, cp_ref, bp_ref, stats_ref):
    stats_ref[0:1, :] = jnp.zeros((1, KP), jnp.float32)
    stats_ref[1:2, :] = jnp.sum(cp_ref[...], axis=0, keepdims=True)
    stats_ref[2:3, :] = jnp.max(qp_ref[...], axis=0, keepdims=True)
    stats_ref[3:4, :] = jnp.sum(bp_ref[...], axis=0, keepdims=True)
    stats_ref[4:8, :] = jnp.zeros((4, KP), jnp.float32)


@functools.partial(
    pl.kernel,
    out_type=jax.ShapeDtypeStruct((NWORK * KP,), jnp.int32),
    mesh=_SC_MESH,
    compiler_params=_SC_PARAMS,
    scratch_types=[
        pltpu.VMEM((CH,), jnp.float32),
        pltpu.VMEM((CH,), jnp.int32),
        pltpu.VMEM((BINS,), jnp.float32),
        pltpu.VMEM((BINS,), jnp.int32),
    ],
)
def _sc_stage_b(q_hbm, id_hbm, qmaxg_hbm, idx_out, q_v, id_v, qg_v, ix_v):
    wid = lax.axis_index("s") * 2 + lax.axis_index("c")
    base = wid * CH
    pltpu.sync_copy(q_hbm.at[pl.ds(base, CH)], q_v)
    pltpu.sync_copy(id_hbm.at[pl.ds(base, CH)], id_v)
    pltpu.sync_copy(qmaxg_hbm, qg_v.at[pl.ds(0, KP)])

    neg1 = jnp.full((16,), -1.0, jnp.float32)
    big = jnp.full((16,), BIG_I32, jnp.int32)
    qg_v[pl.ds(KP, 16)] = neg1

    def init(i, carry):
        ix_v[pl.ds(i * 16, 16)] = big
        return carry
    lax.fori_loop(0, BINS // 16, init, 0)

    lane = lax.iota(jnp.int32, 16)

    def chunk(c, carry):
        sl = pl.ds(c * 16, 16)
        ids_raw = id_v[sl]
        qs = q_v[sl]
        valid = ids_raw >= 0
        ids = jnp.where(valid, ids_raw, DUMMY)
        qmg = plsc.load_gather(qg_v, [ids])
        eq = valid & (qs == qmg)
        gi = lane + (base + c * 16)
        cur = plsc.load_gather(ix_v, [ids])
        win = eq & (gi < cur)

        def cond(w):
            return jnp.any(w)

        def body(w):
            idw = jnp.where(w, ids, DUMMY)
            plsc.store_scatter(ix_v, [idw], gi)
            cur2 = plsc.load_gather(ix_v, [ids])
            return w & (gi < cur2)

        lax.while_loop(cond, body, win)
        return carry
    lax.fori_loop(0, NCH, chunk, 0)

    pltpu.sync_copy(ix_v.at[pl.ds(0, KP)], idx_out.at[pl.ds(wid * KP, KP)])


def _comb2_body(ip_ref, alpha_ref):
    ig = jnp.min(ip_ref[...], axis=0, keepdims=True)
    a = jnp.where(ig == BIG_I32, 0, ig)
    alpha_ref[...] = jnp.broadcast_to(a, (8, KP))


def _phase2_body(alpha_ref, xf_ref, oid_ref, a_ref, stats_ref,
                 out_ref, xk_ref):
    def gather(k, carry):
        a = alpha_ref[k]
        xk_ref[pl.ds(k, 1), :] = xf_ref[pl.ds(a, 1), :]
        return carry
    lax.fori_loop(0, KP, gather, 0)

    cnt = stats_ref[1:2, :]
    qmax = stats_ref[2:3, :]
    nb = stats_ref[3, 0]
    nc = stats_ref[1, 0]

    colid = lax.broadcasted_iota(jnp.int32, (1, KP), 1)
    present = (cnt > 0.0) & (colid >= 1)
    k_f = jnp.sum(present.astype(jnp.float32))
    qk = jnp.where(present, qmax, 0.0)
    c_att = qk / ((cnt + EPS) * k_f)
    c_rep = qk / ((jnp.float32(N) - cnt + EPS) * k_f)

    xk = xk_ref[...]
    xkT = xk.T
    xkxk = jnp.sum(xkT * xkT, axis=0, keepdims=True)
    xkT2 = -2.0 * xkT
    cand = lax.broadcasted_iota(jnp.int32, (B, KP), 1)

    def tile(t, carry):
        s_rep, s_att = carry
        xt = xf_ref[pl.ds(t * B, B), :]
        ids = oid_ref[t][:, None]
        a = a_ref[t][:, None]
        xx = jnp.sum(xt * xt, axis=1)[:, None]
        cross = lax.dot_general(
            xt, xkT2, (((1,), (0,)), ((), ())),
            preferred_element_type=jnp.float32)
        d2p = jnp.maximum(xx + (xkxk + cross), 1e-12)
        dist = d2p * lax.rsqrt(d2p)
        attm = ids == cand
        rep_e = jnp.where(attm, 0.0, jnp.maximum(1.0 - dist, 0.0))
        att_e = jnp.where(attm, d2p, 0.0)
        s_rep = s_rep + jnp.sum(a * rep_e, axis=0, keepdims=True)
        s_att = s_att + jnp.sum(a * att_e, axis=0, keepdims=True)
        return s_rep, s_att

    z = jnp.zeros((1, KP), jnp.float32)
    s_rep, s_att = lax.fori_loop(0, TB, tile, (z, z))

    v_att = jnp.sum(s_att * c_att)
    v_rep = jnp.sum(s_rep * c_rep)
    beta_k = jnp.tanh(jnp.sqrt(jnp.maximum(qmax - Q_MIN, 0.0)))
    l_cow = jnp.sum(jnp.where(present, 1.0 - beta_k, 0.0)) / k_f
    l_noise = nb / jnp.maximum(nc, 1.0)

    li = lax.broadcasted_iota(jnp.int32, (8, 128), 1)
    out = jnp.where(li == 0, v_att,
          jnp.where(li == 1, v_rep,
          jnp.where(li == 2, l_cow,
          jnp.where(li == 3, l_noise, 0.0))))
    out_ref[...] = out


@jax.jit
def kernel(beta, x, object_id, weights):
    beta2 = beta.reshape(TB, B)
    oid2 = object_id.reshape(TB, B)
    w2 = weights.reshape(TB, B)

    q2, a2 = pl.pallas_call(
        _phase0_body,
        out_shape=(
            jax.ShapeDtypeStruct((TB, B), jnp.float32),
            jax.ShapeDtypeStruct((TB, B), jnp.float32),
        ),
    )(beta2, w2)

    pad = PAD_N - N
    qflat = jnp.concatenate([q2.reshape(N), jnp.zeros((pad,), jnp.float32)])
    idflat = jnp.concatenate(
        [object_id, jnp.full((pad,), -1, jnp.int32)])
    bflat = jnp.concatenate([beta, jnp.zeros((pad,), jnp.float32)])

    qmax_p, cnt_p, bsum_p = _sc_stage_a(qflat, idflat, bflat)

    stats = pl.pallas_call(
        _comb1_body,
        out_shape=jax.ShapeDtypeStruct((8, KP), jnp.float32),
    )(qmax_p.reshape(NWORK, KP), cnt_p.reshape(NWORK, KP),
      bsum_p.reshape(NWORK, KP))

    qmaxg = stats[2]

    idx_p = _sc_stage_b(qflat, idflat, qmaxg)

    alpha8 = pl.pallas_call(
        _comb2_body,
        out_shape=jax.ShapeDtypeStruct((8, KP), jnp.int32),
    )(idx_p.reshape(NWORK, KP))
    alphas = alpha8[0]

    out = pl.pallas_call(
        _phase2_body,
        out_shape=jax.ShapeDtypeStruct((8, 128), jnp.float32),
        in_specs=[
            pl.BlockSpec(memory_space=pltpu.MemorySpace.SMEM),
            pl.BlockSpec(memory_space=pltpu.MemorySpace.VMEM),
            pl.BlockSpec(memory_space=pltpu.MemorySpace.VMEM),
            pl.BlockSpec(memory_space=pltpu.MemorySpace.VMEM),
            pl.BlockSpec(memory_space=pltpu.MemorySpace.VMEM),
        ],
        scratch_shapes=[pltpu.VMEM((KP, D), jnp.float32)],
    )(alphas, x, oid2, a2, stats)

    return (out[0, 0], out[0, 1], out[0, 2], out[0, 3])


# speedup vs baseline: 1.3101x; 1.0037x over previous
import functools

import jax
import jax.numpy as jnp
from jax import lax
from jax.experimental import pallas as pl
from jax.experimental.pallas import tpu as pltpu
from jax.experimental.pallas import tpu_sc as plsc

Q_MIN = 0.1
EPS = 1e-09
N = 50000
D = 8
KP = 512
B = 2500
TB = N // B
BIG_I32 = 2**30

NWORK = 32
BINS = KP + 16
DUMMY = KP
CH = 1568
PAD_N = NWORK * CH
NCH = CH // 16

_SC_MESH = plsc.VectorSubcoreMesh(
    core_axis_name="c", subcore_axis_name="s", num_cores=2, num_subcores=16)
_SC_PARAMS = pltpu.CompilerParams(needs_layout_passes=False)


def _phase0_body(beta_ref, w_ref, q_ref, a_ref):
    b = beta_ref[...]
    q = 0.5 * jnp.log((1.0 + b) / (1.0 - b))
    q = q * q + Q_MIN
    q_ref[...] = q
    a_ref[...] = w_ref[...] * q


@functools.partial(
    pl.kernel,
    out_type=(
        jax.ShapeDtypeStruct((NWORK * KP,), jnp.float32),
        jax.ShapeDtypeStruct((NWORK * KP,), jnp.float32),
        jax.ShapeDtypeStruct((NWORK * KP,), jnp.float32),
    ),
    mesh=_SC_MESH,
    compiler_params=_SC_PARAMS,
    scratch_types=[
        pltpu.VMEM((CH,), jnp.float32),
        pltpu.VMEM((CH,), jnp.int32),
        pltpu.VMEM((CH,), jnp.float32),
        pltpu.VMEM((BINS,), jnp.float32),
        pltpu.VMEM((BINS,), jnp.float32),
        pltpu.VMEM((BINS,), jnp.float32),
    ],
)
def _sc_stage_a(q_hbm, id_hbm, beta_hbm, qmax_out, cnt_out, bsum_out,
                q_v, id_v, b_v, qm_v, ct_v, bs_v):
    wid = lax.axis_index("s") * 2 + lax.axis_index("c")
    base = wid * CH
    pltpu.sync_copy(q_hbm.at[pl.ds(base, CH)], q_v)
    pltpu.sync_copy(id_hbm.at[pl.ds(base, CH)], id_v)
    pltpu.sync_copy(beta_hbm.at[pl.ds(base, CH)], b_v)

    neg1 = jnp.full((16,), -1.0, jnp.float32)
    zero = jnp.zeros((16,), jnp.float32)

    def init(i, carry):
        sl = pl.ds(i * 16, 16)
        qm_v[sl] = neg1
        ct_v[sl] = zero
        bs_v[sl] = zero
        return carry
    lax.fori_loop(0, BINS // 16, init, 0)

    ones = jnp.ones((16,), jnp.float32)

    def chunk(c, carry):
        sl = pl.ds(c * 16, 16)
        ids_raw = id_v[sl]
        qs = q_v[sl]
        bs = b_v[sl]
        valid = ids_raw >= 0
        ids = jnp.where(valid, ids_raw, DUMMY)
        plsc.addupdate_scatter(ct_v, [ids], jnp.where(valid, ones, 0.0))
        plsc.addupdate_scatter(bs_v, [ids], jnp.where(valid, bs, 0.0))
        cur = plsc.load_gather(qm_v, [ids])
        win = valid & (qs > cur)

        def cond(w):
            return jnp.any(w)

        def body(w):
            idw = jnp.where(w, ids, DUMMY)
            plsc.store_scatter(qm_v, [idw], qs)
            cur2 = plsc.load_gather(qm_v, [ids])
            return w & (qs > cur2)

        lax.while_loop(cond, body, win)
        return carry
    lax.fori_loop(0, NCH, chunk, 0)

    pltpu.sync_copy(qm_v.at[pl.ds(0, KP)], qmax_out.at[pl.ds(wid * KP, KP)])
    pltpu.sync_copy(ct_v.at[pl.ds(0, KP)], cnt_out.at[pl.ds(wid * KP, KP)])
    pltpu.sync_copy(bs_v.at[pl.ds(0, KP)], bsum_out.at[pl.ds(wid * KP, KP)])


def _comb1_body(qp_ref, cp_ref, bp_ref, stats_ref):
    stats_ref[0:1, :] = jnp.zeros((1, KP), jnp.float32)
    stats_ref[1:2, :] = jnp.sum(cp_ref[...], axis=0, keepdims=True)
    stats_ref[2:3, :] = jnp.max(qp_ref[...], axis=0, keepdims=True)
    stats_ref[3:4, :] = jnp.sum(bp_ref[...], axis=0, keepdims=True)
    stats_ref[4:8, :] = jnp.zeros((4, KP), jnp.float32)


@functools.partial(
    pl.kernel,
    out_type=jax.ShapeDtypeStruct((NWORK * KP,), jnp.int32),
    mesh=_SC_MESH,
    compiler_params=_SC_PARAMS,
    scratch_types=[
        pltpu.VMEM((CH,), jnp.float32),
        pltpu.VMEM((CH,), jnp.int32),
        pltpu.VMEM((BINS,), jnp.float32),
        pltpu.VMEM((BINS,), jnp.int32),
    ],
)
def _sc_stage_b(q_hbm, id_hbm, qmaxg_hbm, idx_out, q_v, id_v, qg_v, ix_v):
    wid = lax.axis_index("s") * 2 + lax.axis_index("c")
    base = wid * CH
    pltpu.sync_copy(q_hbm.at[pl.ds(base, CH)], q_v)
    pltpu.sync_copy(id_hbm.at[pl.ds(base, CH)], id_v)
    pltpu.sync_copy(qmaxg_hbm, qg_v.at[pl.ds(0, KP)])

    neg1 = jnp.full((16,), -1.0, jnp.float32)
    big = jnp.full((16,), BIG_I32, jnp.int32)
    qg_v[pl.ds(KP, 16)] = neg1

    def init(i, carry):
        ix_v[pl.ds(i * 16, 16)] = big
        return carry
    lax.fori_loop(0, BINS // 16, init, 0)

    lane = lax.iota(jnp.int32, 16)

    def chunk(c, carry):
        sl = pl.ds(c * 16, 16)
        ids_raw = id_v[sl]
        qs = q_v[sl]
        valid = ids_raw >= 0
        ids = jnp.where(valid, ids_raw, DUMMY)
        qmg = plsc.load_gather(qg_v, [ids])
        eq = valid & (qs == qmg)
        gi = lane + (base + c * 16)
        cur = plsc.load_gather(ix_v, [ids])
        win = eq & (gi < cur)

        def cond(w):
            return jnp.any(w)

        def body(w):
            idw = jnp.where(w, ids, DUMMY)
            plsc.store_scatter(ix_v, [idw], gi)
            cur2 = plsc.load_gather(ix_v, [ids])
            return w & (gi < cur2)

        lax.while_loop(cond, body, win)
        return carry
    lax.fori_loop(0, NCH, chunk, 0)

    pltpu.sync_copy(ix_v.at[pl.ds(0, KP)], idx_out.at[pl.ds(wid * KP, KP)])


def _comb2_body(ip_ref, alpha_ref):
    ig = jnp.min(ip_ref[...], axis=0, keepdims=True)
    a = jnp.where(ig == BIG_I32, 0, ig)
    alpha_ref[...] = jnp.broadcast_to(a, (8, KP))


def _phase2_body(alpha_ref, xf_ref, oid_ref, a_ref, stats_ref,
                 out_ref, xk_ref):
    def gather(k, carry):
        a = alpha_ref[k]
        xk_ref[pl.ds(k, 1), :] = xf_ref[pl.ds(a, 1), :]
        return carry
    lax.fori_loop(0, KP, gather, 0)

    cnt = stats_ref[1:2, :]
    qmax = stats_ref[2:3, :]
    nb = stats_ref[3, 0]
    nc = stats_ref[1, 0]

    colid = lax.broadcasted_iota(jnp.int32, (1, KP), 1)
    present = (cnt > 0.0) & (colid >= 1)
    k_f = jnp.sum(present.astype(jnp.float32))
    qk = jnp.where(present, qmax, 0.0)
    c_att = qk / ((cnt + EPS) * k_f)
    c_rep = qk / ((jnp.float32(N) - cnt + EPS) * k_f)

    xk = xk_ref[...]
    xkT = xk.T
    xkxk = jnp.sum(xkT * xkT, axis=0, keepdims=True)
    xkT2 = -2.0 * xkT
    cand = lax.broadcasted_iota(jnp.int32, (B, KP), 1)

    def tile(t, carry):
        s_rep, s_att = carry
        xt = xf_ref[pl.ds(t * B, B), :]
        ids = oid_ref[t][:, None]
        a = a_ref[t][:, None]
        xx = jnp.sum(xt * xt, axis=1)[:, None]
        cross = lax.dot_general(
            xt, xkT2, (((1,), (0,)), ((), ())),
            preferred_element_type=jnp.float32)
        d2p = jnp.maximum(xx + (xkxk + cross), 1e-12)
        dist = d2p * lax.rsqrt(d2p)
        attm = ids == cand
        rep_e = jnp.where(attm, 0.0, jnp.maximum(1.0 - dist, 0.0))
        att_e = jnp.where(attm, d2p, 0.0)
        s_rep = s_rep + jnp.sum(a * rep_e, axis=0, keepdims=True)
        s_att = s_att + jnp.sum(a * att_e, axis=0, keepdims=True)
        return s_rep, s_att

    z = jnp.zeros((1, KP), jnp.float32)
    s_rep, s_att = lax.fori_loop(0, TB, tile, (z, z))

    v_att = jnp.sum(s_att * c_att)
    v_rep = jnp.sum(s_rep * c_rep)
    beta_k = jnp.tanh(jnp.sqrt(jnp.maximum(qmax - Q_MIN, 0.0)))
    l_cow = jnp.sum(jnp.where(present, 1.0 - beta_k, 0.0)) / k_f
    l_noise = nb / jnp.maximum(nc, 1.0)

    li = lax.broadcasted_iota(jnp.int32, (8, 128), 1)
    out = jnp.where(li == 0, v_att,
          jnp.where(li == 1, v_rep,
          jnp.where(li == 2, l_cow,
          jnp.where(li == 3, l_noise, 0.0))))
    out_ref[...] = out


@jax.jit
def kernel(beta, x, object_id, weights):
    beta2 = beta.reshape(TB, B)
    oid2 = object_id.reshape(TB, B)
    w2 = weights.reshape(TB, B)

    q2, a2 = pl.pallas_call(
        _phase0_body,
        out_shape=(
            jax.ShapeDtypeStruct((TB, B), jnp.float32),
            jax.ShapeDtypeStruct((TB, B), jnp.float32),
        ),
    )(beta2, w2)

    pad = PAD_N - N
    qflat = jnp.concatenate([q2.reshape(N), jnp.zeros((pad,), jnp.float32)])
    idflat = jnp.concatenate(
        [object_id, jnp.full((pad,), -1, jnp.int32)])
    bflat = jnp.concatenate([beta, jnp.zeros((pad,), jnp.float32)])

    qmax_p, cnt_p, bsum_p = _sc_stage_a(qflat, idflat, bflat)

    stats = pl.pallas_call(
        _comb1_body,
        out_shape=jax.ShapeDtypeStruct((8, KP), jnp.float32),
    )(qmax_p.reshape(NWORK, KP), cnt_p.reshape(NWORK, KP),
      bsum_p.reshape(NWORK, KP))

    qmaxg = stats[2]

    idx_p = _sc_stage_b(qflat, idflat, qmaxg)

    alpha8 = pl.pallas_call(
        _comb2_body,
        out_shape=jax.ShapeDtypeStruct((8, KP), jnp.int32),
    )(idx_p.reshape(NWORK, KP))
    alphas = alpha8[0]

    out = pl.pallas_call(
        _phase2_body,
        out_shape=jax.ShapeDtypeStruct((8, 128), jnp.float32),
        in_specs=[
            pl.BlockSpec(memory_space=pltpu.MemorySpace.SMEM),
            pl.BlockSpec(memory_space=pltpu.MemorySpace.VMEM),
            pl.BlockSpec(memory_space=pltpu.MemorySpace.VMEM),
            pl.BlockSpec(memory_space=pltpu.MemorySpace.VMEM),
            pl.BlockSpec(memory_space=pltpu.MemorySpace.VMEM),
        ],
        scratch_shapes=[pltpu.VMEM((KP, D), jnp.float32)],
    )(alphas, x, oid2, a2, stats)

    return (out[0, 0], out[0, 1], out[0, 2], out[0, 3])


# speedup vs baseline: 1.3234x; 1.0101x over previous
import functools

import jax
import jax.numpy as jnp
from jax import lax
from jax.experimental import pallas as pl
from jax.experimental.pallas import tpu as pltpu
from jax.experimental.pallas import tpu_sc as plsc

Q_MIN = 0.1
EPS = 1e-09
N = 50000
D = 8
KP = 512
B = 5000
TB = N // B
BIG_I32 = 2**30

NWORK = 32
BINS = KP + 16
DUMMY = KP
CH = 1568
PAD_N = NWORK * CH
NCH = CH // 16

_SC_MESH = plsc.VectorSubcoreMesh(
    core_axis_name="c", subcore_axis_name="s", num_cores=2, num_subcores=16)
_SC_PARAMS = pltpu.CompilerParams(needs_layout_passes=False)


def _phase0_body(beta_ref, w_ref, q_ref, a_ref):
    b = beta_ref[...]
    q = 0.5 * jnp.log((1.0 + b) / (1.0 - b))
    q = q * q + Q_MIN
    q_ref[...] = q
    a_ref[...] = w_ref[...] * q


@functools.partial(
    pl.kernel,
    out_type=(
        jax.ShapeDtypeStruct((NWORK * KP,), jnp.float32),
        jax.ShapeDtypeStruct((NWORK * KP,), jnp.float32),
        jax.ShapeDtypeStruct((NWORK * KP,), jnp.float32),
    ),
    mesh=_SC_MESH,
    compiler_params=_SC_PARAMS,
    scratch_types=[
        pltpu.VMEM((CH,), jnp.float32),
        pltpu.VMEM((CH,), jnp.int32),
        pltpu.VMEM((CH,), jnp.float32),
        pltpu.VMEM((BINS,), jnp.float32),
        pltpu.VMEM((BINS,), jnp.float32),
        pltpu.VMEM((BINS,), jnp.float32),
    ],
)
def _sc_stage_a(q_hbm, id_hbm, beta_hbm, qmax_out, cnt_out, bsum_out,
                q_v, id_v, b_v, qm_v, ct_v, bs_v):
    wid = lax.axis_index("s") * 2 + lax.axis_index("c")
    base = wid * CH
    pltpu.sync_copy(q_hbm.at[pl.ds(base, CH)], q_v)
    pltpu.sync_copy(id_hbm.at[pl.ds(base, CH)], id_v)
    pltpu.sync_copy(beta_hbm.at[pl.ds(base, CH)], b_v)

    neg1 = jnp.full((16,), -1.0, jnp.float32)
    zero = jnp.zeros((16,), jnp.float32)

    def init(i, carry):
        sl = pl.ds(i * 16, 16)
        qm_v[sl] = neg1
        ct_v[sl] = zero
        bs_v[sl] = zero
        return carry
    lax.fori_loop(0, BINS // 16, init, 0)

    ones = jnp.ones((16,), jnp.float32)

    def chunk(c, carry):
        sl = pl.ds(c * 16, 16)
        ids_raw = id_v[sl]
        qs = q_v[sl]
        bs = b_v[sl]
        valid = ids_raw >= 0
        ids = jnp.where(valid, ids_raw, DUMMY)
        plsc.addupdate_scatter(ct_v, [ids], jnp.where(valid, ones, 0.0))
        plsc.addupdate_scatter(bs_v, [ids], jnp.where(valid, bs, 0.0))
        cur = plsc.load_gather(qm_v, [ids])
        win = valid & (qs > cur)

        def cond(w):
            return jnp.any(w)

        def body(w):
            idw = jnp.where(w, ids, DUMMY)
            plsc.store_scatter(qm_v, [idw], qs)
            cur2 = plsc.load_gather(qm_v, [ids])
            return w & (qs > cur2)

        lax.while_loop(cond, body, win)
        return carry
    lax.fori_loop(0, NCH, chunk, 0)

    pltpu.sync_copy(qm_v.at[pl.ds(0, KP)], qmax_out.at[pl.ds(wid * KP, KP)])
    pltpu.sync_copy(ct_v.at[pl.ds(0, KP)], cnt_out.at[pl.ds(wid * KP, KP)])
    pltpu.sync_copy(bs_v.at[pl.ds(0, KP)], bsum_out.at[pl.ds(wid * KP, KP)])


def _comb1_body(qp_ref, cp_ref, bp_ref, stats_ref):
    stats_ref[0:1, :] = jnp.zeros((1, KP), jnp.float32)
    stats_ref[1:2, :] = jnp.sum(cp_ref[...], axis=0, keepdims=True)
    stats_ref[2:3, :] = jnp.max(qp_ref[...], axis=0, keepdims=True)
    stats_ref[3:4, :] = jnp.sum(bp_ref[...], axis=0, keepdims=True)
    stats_ref[4:8, :] = jnp.zeros((4, KP), jnp.float32)


@functools.partial(
    pl.kernel,
    out_type=jax.ShapeDtypeStruct((NWORK * KP,), jnp.int32),
    mesh=_SC_MESH,
    compiler_params=_SC_PARAMS,
    scratch_types=[
        pltpu.VMEM((CH,), jnp.float32),
        pltpu.VMEM((CH,), jnp.int32),
        pltpu.VMEM((BINS,), jnp.float32),
        pltpu.VMEM((BINS,), jnp.int32),
    ],
)
def _sc_stage_b(q_hbm, id_hbm, qmaxg_hbm, idx_out, q_v, id_v, qg_v, ix_v):
    wid = lax.axis_index("s") * 2 + lax.axis_index("c")
    base = wid * CH
    pltpu.sync_copy(q_hbm.at[pl.ds(base, CH)], q_v)
    pltpu.sync_copy(id_hbm.at[pl.ds(base, CH)], id_v)
    pltpu.sync_copy(qmaxg_hbm, qg_v.at[pl.ds(0, KP)])

    neg1 = jnp.full((16,), -1.0, jnp.float32)
    big = jnp.full((16,), BIG_I32, jnp.int32)
    qg_v[pl.ds(KP, 16)] = neg1

    def init(i, carry):
        ix_v[pl.ds(i * 16, 16)] = big
        return carry
    lax.fori_loop(0, BINS // 16, init, 0)

    lane = lax.iota(jnp.int32, 16)

    def chunk(c, carry):
        sl = pl.ds(c * 16, 16)
        ids_raw = id_v[sl]
        qs = q_v[sl]
        valid = ids_raw >= 0
        ids = jnp.where(valid, ids_raw, DUMMY)
        qmg = plsc.load_gather(qg_v, [ids])
        eq = valid & (qs == qmg)
        gi = lane + (base + c * 16)
        cur = plsc.load_gather(ix_v, [ids])
        win = eq & (gi < cur)

        def cond(w):
            return jnp.any(w)

        def body(w):
            idw = jnp.where(w, ids, DUMMY)
            plsc.store_scatter(ix_v, [idw], gi)
            cur2 = plsc.load_gather(ix_v, [ids])
            return w & (gi < cur2)

        lax.while_loop(cond, body, win)
        return carry
    lax.fori_loop(0, NCH, chunk, 0)

    pltpu.sync_copy(ix_v.at[pl.ds(0, KP)], idx_out.at[pl.ds(wid * KP, KP)])


def _comb2_body(ip_ref, alpha_ref):
    ig = jnp.min(ip_ref[...], axis=0, keepdims=True)
    a = jnp.where(ig == BIG_I32, 0, ig)
    alpha_ref[...] = jnp.broadcast_to(a, (8, KP))


def _phase2_body(alpha_ref, xf_ref, oid_ref, a_ref, stats_ref,
                 out_ref, xk_ref):
    def gather(k, carry):
        a = alpha_ref[k]
        xk_ref[pl.ds(k, 1), :] = xf_ref[pl.ds(a, 1), :]
        return carry
    lax.fori_loop(0, KP, gather, 0)

    cnt = stats_ref[1:2, :]
    qmax = stats_ref[2:3, :]
    nb = stats_ref[3, 0]
    nc = stats_ref[1, 0]

    colid = lax.broadcasted_iota(jnp.int32, (1, KP), 1)
    present = (cnt > 0.0) & (colid >= 1)
    k_f = jnp.sum(present.astype(jnp.float32))
    qk = jnp.where(present, qmax, 0.0)
    c_att = qk / ((cnt + EPS) * k_f)
    c_rep = qk / ((jnp.float32(N) - cnt + EPS) * k_f)

    xk = xk_ref[...]
    xkT = xk.T
    xkxk = jnp.sum(xkT * xkT, axis=0, keepdims=True)
    xkT2 = -2.0 * xkT
    cand = lax.broadcasted_iota(jnp.int32, (B, KP), 1)

    def tile(t, carry):
        s_rep, s_att = carry
        xt = xf_ref[pl.ds(t * B, B), :]
        ids = oid_ref[t][:, None]
        a = a_ref[t][:, None]
        xx = jnp.sum(xt * xt, axis=1)[:, None]
        cross = lax.dot_general(
            xt, xkT2, (((1,), (0,)), ((), ())),
            preferred_element_type=jnp.float32)
        d2p = jnp.maximum(xx + (xkxk + cross), 1e-12)
        dist = d2p * lax.rsqrt(d2p)
        attm = ids == cand
        rep_e = jnp.where(attm, 0.0, jnp.maximum(1.0 - dist, 0.0))
        att_e = jnp.where(attm, d2p, 0.0)
        s_rep = s_rep + jnp.sum(a * rep_e, axis=0, keepdims=True)
        s_att = s_att + jnp.sum(a * att_e, axis=0, keepdims=True)
        return s_rep, s_att

    z = jnp.zeros((1, KP), jnp.float32)
    s_rep, s_att = lax.fori_loop(0, TB, tile, (z, z))

    v_att = jnp.sum(s_att * c_att)
    v_rep = jnp.sum(s_rep * c_rep)
    beta_k = jnp.tanh(jnp.sqrt(jnp.maximum(qmax - Q_MIN, 0.0)))
    l_cow = jnp.sum(jnp.where(present, 1.0 - beta_k, 0.0)) / k_f
    l_noise = nb / jnp.maximum(nc, 1.0)

    li = lax.broadcasted_iota(jnp.int32, (8, 128), 1)
    out = jnp.where(li == 0, v_att,
          jnp.where(li == 1, v_rep,
          jnp.where(li == 2, l_cow,
          jnp.where(li == 3, l_noise, 0.0))))
    out_ref[...] = out


@jax.jit
def kernel(beta, x, object_id, weights):
    beta2 = beta.reshape(TB, B)
    oid2 = object_id.reshape(TB, B)
    w2 = weights.reshape(TB, B)

    q2, a2 = pl.pallas_call(
        _phase0_body,
        out_shape=(
            jax.ShapeDtypeStruct((TB, B), jnp.float32),
            jax.ShapeDtypeStruct((TB, B), jnp.float32),
        ),
    )(beta2, w2)

    pad = PAD_N - N
    qflat = jnp.concatenate([q2.reshape(N), jnp.zeros((pad,), jnp.float32)])
    idflat = jnp.concatenate(
        [object_id, jnp.full((pad,), -1, jnp.int32)])
    bflat = jnp.concatenate([beta, jnp.zeros((pad,), jnp.float32)])

    qmax_p, cnt_p, bsum_p = _sc_stage_a(qflat, idflat, bflat)

    stats = pl.pallas_call(
        _comb1_body,
        out_shape=jax.ShapeDtypeStruct((8, KP), jnp.float32),
    )(qmax_p.reshape(NWORK, KP), cnt_p.reshape(NWORK, KP),
      bsum_p.reshape(NWORK, KP))

    qmaxg = stats[2]

    idx_p = _sc_stage_b(qflat, idflat, qmaxg)

    alpha8 = pl.pallas_call(
        _comb2_body,
        out_shape=jax.ShapeDtypeStruct((8, KP), jnp.int32),
    )(idx_p.reshape(NWORK, KP))
    alphas = alpha8[0]

    out = pl.pallas_call(
        _phase2_body,
        out_shape=jax.ShapeDtypeStruct((8, 128), jnp.float32),
        in_specs=[
            pl.BlockSpec(memory_space=pltpu.MemorySpace.SMEM),
            pl.BlockSpec(memory_space=pltpu.MemorySpace.VMEM),
            pl.BlockSpec(memory_space=pltpu.MemorySpace.VMEM),
            pl.BlockSpec(memory_space=pltpu.MemorySpace.VMEM),
            pl.BlockSpec(memory_space=pltpu.MemorySpace.VMEM),
        ],
        scratch_shapes=[pltpu.VMEM((KP, D), jnp.float32)],
    )(alphas, x, oid2, a2, stats)

    return (out[0, 0], out[0, 1], out[0, 2], out[0, 3])
